# baseline (speedup 1.0000x reference)
import jax
import jax.numpy as jnp
from jax import lax
from jax.experimental import pallas as pl
from jax.experimental.pallas import tpu as pltpu
from jax.experimental.pallas import tpu_sc as plsc

D_MODEL = 64
SCALE = 8.0
CHUNK = 128
LANES = 16
NW = 32


def _wid():
    return lax.axis_index("s") * 2 + lax.axis_index("c")


def _lookup_body(idx_hbm, tableP_hbm, out_hbm, idx_v, gbufs, obufs, pairbufs,
                 gsems, osems, half):
    wid = _wid()
    n_chunks = idx_hbm.shape[1]
    base = wid * n_chunks * CHUNK

    pltpu.sync_copy(idx_hbm.at[wid], idx_v)

    def prep_pair(c, b):
        for k in range(CHUNK // LANES):
            v = idx_v[c, pl.ds(LANES * k, LANES)]
            pairbufs[b][pl.ds(LANES * k, LANES)] = jnp.where(
                v >= half, v - half, v)

    def gather(b):
        return pltpu.make_async_copy(tableP_hbm.at[pairbufs[b]], gbufs[b],
                                     gsems[b])

    def copy_out(c, b):
        return pltpu.make_async_copy(
            obufs[b], out_hbm.at[pl.ds(base + c * CHUNK, CHUNK)], osems[b])

    def select(c, b):
        @plsc.parallel_loop(0, CHUNK // LANES, unroll=2)
        def _(k):
            iv = idx_v[c, pl.ds(k * LANES, LANES)]
            offs = jnp.where(iv >= half, D_MODEL, 0).astype(jnp.int32)
            for j in range(LANES):
                r = k * LANES + j
                off = offs[j]
                for l in range(D_MODEL // LANES):
                    obufs[b][r, pl.ds(l * LANES, LANES)] = (
                        gbufs[b][r, pl.ds(off + l * LANES, LANES)])

    for b in range(2):
        prep_pair(b, b)
        gather(b).start()
    for b in range(2):
        gather(b).wait()
        select(b, b)
        copy_out(b, b).start()
        prep_pair(b + 2, b)
        gather(b).start()

    def group(i, carry):
        for b in range(2):
            c = 2 * i + b
            gather(b).wait()
            copy_out(c - 2, b).wait()
            select(c, b)
            copy_out(c, b).start()
            prep_pair(c + 2, b)
            gather(b).start()
        return carry

    lax.fori_loop(1, n_chunks // 2 - 1, group, 0)

    for b in range(2):
        c = n_chunks - 2 + b
        gather(b).wait()
        copy_out(c - 2, b).wait()
        select(c, b)
        copy_out(c, b).start()
    for b in range(2):
        copy_out(n_chunks - 2 + b, b).wait()


def kernel(x, embed_weight):
    n_x_rows, row_len = x.shape
    n = n_x_rows * row_len
    v = embed_weight.shape[0]
    xi = x.astype(jnp.int32)
    idx3 = xi.reshape(NW, n // (NW * CHUNK), CHUNK)
    tableT = embed_weight.T

    mesh = plsc.VectorSubcoreMesh(core_axis_name="c", subcore_axis_name="s")
    params = pltpu.CompilerParams(needs_layout_passes=False)
    blk = 256
    vp2 = (v // 2 + blk - 1) // blk * blk

    n_blk = vp2 // blk

    def a_tc_body(x1_ref, x2_ref, o_ref):
        eye = jnp.eye(D_MODEL, dtype=jnp.float32) * SCALE
        dims = (((0,), (0,)), ((), ()))
        left = jax.lax.dot_general(x1_ref[...], eye, dims,
                                   preferred_element_type=jnp.float32)
        right = jax.lax.dot_general(x2_ref[...], eye, dims,
                                    preferred_element_type=jnp.float32)
        o_ref[...] = jnp.concatenate([left, right], axis=1)

    last_in_blk = (v - 1) // blk

    repack = pl.pallas_call(
        a_tc_body,
        grid=(n_blk,),
        in_specs=[
            pl.BlockSpec((D_MODEL, blk), lambda i: (0, i)),
            pl.BlockSpec((D_MODEL, blk),
                         lambda i: (0, jnp.minimum(i + n_blk, last_in_blk))),
        ],
        out_specs=pl.BlockSpec((blk, 2 * D_MODEL), lambda i: (i, 0)),
        out_shape=jax.ShapeDtypeStruct((vp2, 2 * D_MODEL), jnp.float32),
    )
    tableP = repack(tableT, tableT)

    def b_body(idx_hbm, tableP_hbm, out_hbm, idx_v, *scratch):
        _lookup_body(idx_hbm, tableP_hbm, out_hbm, idx_v, scratch[0:2],
                     scratch[2:4], scratch[4:6], scratch[6:8], scratch[8:10],
                     vp2)

    lookup = pl.kernel(
        b_body,
        out_type=jax.ShapeDtypeStruct((n, D_MODEL), jnp.float32),
        mesh=mesh,
        scratch_types=(
            [pltpu.VMEM((n // (NW * CHUNK), CHUNK), jnp.int32)]
            + [pltpu.VMEM((CHUNK, 2 * D_MODEL), jnp.float32) for _ in range(2)]
            + [pltpu.VMEM((CHUNK, D_MODEL), jnp.float32) for _ in range(2)]
            + [pltpu.VMEM((CHUNK,), jnp.int32) for _ in range(2)]
            + [pltpu.SemaphoreType.DMA for _ in range(4)]
        ),
        compiler_params=params,
    )
    out = lookup(idx3, tableP)
    return out.reshape(n_x_rows, row_len, D_MODEL)


# speedup vs baseline: 2.3391x; 2.3391x over previous
import jax
import jax.numpy as jnp
from jax import lax
from jax.experimental import pallas as pl
from jax.experimental.pallas import tpu as pltpu
from jax.experimental.pallas import tpu_sc as plsc

D_MODEL = 64
SCALE = 8.0
CHUNK = 128
LANES = 16
NW = 32


def _wid():
    return lax.axis_index("s") * 2 + lax.axis_index("c")


def _lookup_body(idx_hbm, tableP_hbm, out_hbm, idx_v, gbufs, obufs, pairbufs,
                 gsems, osems, half):
    wid = _wid()
    n_chunks = idx_hbm.shape[1]
    base = wid * n_chunks * CHUNK

    pltpu.sync_copy(idx_hbm.at[wid], idx_v)

    def prep_pair(c, b):
        for k in range(CHUNK // LANES):
            v = idx_v[c, pl.ds(LANES * k, LANES)]
            pairbufs[b][pl.ds(LANES * k, LANES)] = jnp.where(
                v >= half, v - half, v)

    def gather(b):
        return pltpu.make_async_copy(tableP_hbm.at[pairbufs[b]], gbufs[b],
                                     gsems[b])

    def copy_out(c, b):
        return pltpu.make_async_copy(
            obufs[b], out_hbm.at[pl.ds(base + c * CHUNK, CHUNK)], osems[b])

    def select(c, b):
        @plsc.parallel_loop(0, CHUNK // LANES, unroll=2)
        def _(k):
            iv = idx_v[c, pl.ds(k * LANES, LANES)]
            offs = jnp.where(iv >= half, D_MODEL, 0).astype(jnp.int32)
            for j in range(LANES):
                r = k * LANES + j
                off = offs[j]
                for l in range(D_MODEL // LANES):
                    obufs[b][r, pl.ds(l * LANES, LANES)] = (
                        gbufs[b][r, pl.ds(off + l * LANES, LANES)])

    for b in range(2):
        prep_pair(b, b)
        gather(b).start()
    for b in range(2):
        gather(b).wait()
        select(b, b)
        copy_out(b, b).start()
        prep_pair(b + 2, b)
        gather(b).start()

    def group(i, carry):
        for b in range(2):
            c = 2 * i + b
            gather(b).wait()
            copy_out(c - 2, b).wait()
            select(c, b)
            copy_out(c, b).start()
            prep_pair(c + 2, b)
            gather(b).start()
        return carry

    lax.fori_loop(1, n_chunks // 2 - 1, group, 0)

    for b in range(2):
        c = n_chunks - 2 + b
        gather(b).wait()
        copy_out(c - 2, b).wait()
        select(c, b)
        copy_out(c, b).start()
    for b in range(2):
        copy_out(n_chunks - 2 + b, b).wait()


def kernel(x, embed_weight):
    n_x_rows, row_len = x.shape
    n = n_x_rows * row_len
    v = embed_weight.shape[0]
    xi = x.astype(jnp.int32)
    idx3 = xi.reshape(NW, n // (NW * CHUNK), CHUNK)
    tableT = embed_weight.T

    mesh = plsc.VectorSubcoreMesh(core_axis_name="c", subcore_axis_name="s")
    params = pltpu.CompilerParams(needs_layout_passes=False)
    blk = 4096
    vp2 = (v // 2 + blk - 1) // blk * blk

    n_blk = vp2 // blk

    def a_tc_body(x1_ref, x2_ref, o_ref):
        eye = jnp.eye(D_MODEL, dtype=jnp.float32) * SCALE
        dims = (((0,), (0,)), ((), ()))
        left = jax.lax.dot_general(x1_ref[...], eye, dims,
                                   preferred_element_type=jnp.float32)
        right = jax.lax.dot_general(x2_ref[...], eye, dims,
                                    preferred_element_type=jnp.float32)
        o_ref[...] = jnp.concatenate([left, right], axis=1)

    last_in_blk = (v - 1) // blk

    repack = pl.pallas_call(
        a_tc_body,
        grid=(n_blk,),
        in_specs=[
            pl.BlockSpec((D_MODEL, blk), lambda i: (0, i)),
            pl.BlockSpec((D_MODEL, blk),
                         lambda i: (0, jnp.minimum(i + n_blk, last_in_blk))),
        ],
        out_specs=pl.BlockSpec((blk, 2 * D_MODEL), lambda i: (i, 0)),
        out_shape=jax.ShapeDtypeStruct((vp2, 2 * D_MODEL), jnp.float32),
    )
    tableP = repack(tableT, tableT)

    def b_body(idx_hbm, tableP_hbm, out_hbm, idx_v, *scratch):
        _lookup_body(idx_hbm, tableP_hbm, out_hbm, idx_v, scratch[0:2],
                     scratch[2:4], scratch[4:6], scratch[6:8], scratch[8:10],
                     vp2)

    lookup = pl.kernel(
        b_body,
        out_type=jax.ShapeDtypeStruct((n, D_MODEL), jnp.float32),
        mesh=mesh,
        scratch_types=(
            [pltpu.VMEM((n // (NW * CHUNK), CHUNK), jnp.int32)]
            + [pltpu.VMEM((CHUNK, 2 * D_MODEL), jnp.float32) for _ in range(2)]
            + [pltpu.VMEM((CHUNK, D_MODEL), jnp.float32) for _ in range(2)]
            + [pltpu.VMEM((CHUNK,), jnp.int32) for _ in range(2)]
            + [pltpu.SemaphoreType.DMA for _ in range(4)]
        ),
        compiler_params=params,
    )
    out = lookup(idx3, tableP)
    return out.reshape(n_x_rows, row_len, D_MODEL)


# speedup vs baseline: 2.4473x; 1.0463x over previous
import jax
import jax.numpy as jnp
from jax import lax
from jax.experimental import pallas as pl
from jax.experimental.pallas import tpu as pltpu
from jax.experimental.pallas import tpu_sc as plsc

D_MODEL = 64
SCALE = 8.0
CHUNK = 128
LANES = 16
NW = 32


def _wid():
    return lax.axis_index("s") * 2 + lax.axis_index("c")


def _lookup_body(idx_hbm, tableP_hbm, out_hbm, idx_v, gbufs, obufs, pairbufs,
                 gsems, osems, half):
    wid = _wid()
    n_chunks = idx_hbm.shape[1]
    base = wid * n_chunks * CHUNK

    pltpu.sync_copy(idx_hbm.at[wid], idx_v)

    def prep_pair(c, b):
        for k in range(CHUNK // LANES):
            v = idx_v[c, pl.ds(LANES * k, LANES)]
            pairbufs[b][pl.ds(LANES * k, LANES)] = jnp.where(
                v >= half, v - half, v)

    def gather(b):
        return pltpu.make_async_copy(tableP_hbm.at[pairbufs[b]], gbufs[b],
                                     gsems[b])

    def copy_out(c, b):
        return pltpu.make_async_copy(
            obufs[b], out_hbm.at[pl.ds(base + c * CHUNK, CHUNK)], osems[b])

    def select(c, b):
        @plsc.parallel_loop(0, CHUNK // LANES, unroll=2)
        def _(k):
            iv = idx_v[c, pl.ds(k * LANES, LANES)]
            offs = jnp.where(iv >= half, D_MODEL, 0).astype(jnp.int32)
            for j in range(LANES):
                r = k * LANES + j
                off = offs[j]
                for l in range(D_MODEL // LANES):
                    obufs[b][r, pl.ds(l * LANES, LANES)] = (
                        gbufs[b][r, pl.ds(off + l * LANES, LANES)])

    for b in range(2):
        prep_pair(b, b)
        gather(b).start()
    for b in range(2):
        gather(b).wait()
        select(b, b)
        copy_out(b, b).start()
        prep_pair(b + 2, b)
        gather(b).start()

    def group(i, carry):
        for b in range(2):
            c = 2 * i + b
            gather(b).wait()
            copy_out(c - 2, b).wait()
            select(c, b)
            copy_out(c, b).start()
            prep_pair(c + 2, b)
            gather(b).start()
        return carry

    lax.fori_loop(1, n_chunks // 2 - 1, group, 0)

    for b in range(2):
        c = n_chunks - 2 + b
        gather(b).wait()
        copy_out(c - 2, b).wait()
        select(c, b)
        copy_out(c, b).start()
    for b in range(2):
        copy_out(n_chunks - 2 + b, b).wait()


def kernel(x, embed_weight):
    n_x_rows, row_len = x.shape
    n = n_x_rows * row_len
    v = embed_weight.shape[0]
    xi = x.astype(jnp.int32)
    idx3 = xi.reshape(NW, n // (NW * CHUNK), CHUNK)
    tableT = embed_weight.T

    mesh = plsc.VectorSubcoreMesh(core_axis_name="c", subcore_axis_name="s")
    params = pltpu.CompilerParams(needs_layout_passes=False)
    blk = 8192
    vp2 = (v // 2 + blk - 1) // blk * blk

    n_blk = vp2 // blk

    def a_tc_body(x1_ref, x2_ref, o_ref):
        eye = jnp.eye(D_MODEL, dtype=jnp.float32) * SCALE
        dims = (((0,), (0,)), ((), ()))
        left = jax.lax.dot_general(x1_ref[...], eye, dims,
                                   preferred_element_type=jnp.float32)
        right = jax.lax.dot_general(x2_ref[...], eye, dims,
                                    preferred_element_type=jnp.float32)
        o_ref[...] = jnp.concatenate([left, right], axis=1)

    last_in_blk = (v - 1) // blk

    repack = pl.pallas_call(
        a_tc_body,
        grid=(n_blk,),
        in_specs=[
            pl.BlockSpec((D_MODEL, blk), lambda i: (0, i)),
            pl.BlockSpec((D_MODEL, blk),
                         lambda i: (0, jnp.minimum(i + n_blk, last_in_blk))),
        ],
        out_specs=pl.BlockSpec((blk, 2 * D_MODEL), lambda i: (i, 0)),
        out_shape=jax.ShapeDtypeStruct((vp2, 2 * D_MODEL), jnp.float32),
    )
    tableP = repack(tableT, tableT)

    def b_body(idx_hbm, tableP_hbm, out_hbm, idx_v, *scratch):
        _lookup_body(idx_hbm, tableP_hbm, out_hbm, idx_v, scratch[0:2],
                     scratch[2:4], scratch[4:6], scratch[6:8], scratch[8:10],
                     vp2)

    lookup = pl.kernel(
        b_body,
        out_type=jax.ShapeDtypeStruct((n, D_MODEL), jnp.float32),
        mesh=mesh,
        scratch_types=(
            [pltpu.VMEM((n // (NW * CHUNK), CHUNK), jnp.int32)]
            + [pltpu.VMEM((CHUNK, 2 * D_MODEL), jnp.float32) for _ in range(2)]
            + [pltpu.VMEM((CHUNK, D_MODEL), jnp.float32) for _ in range(2)]
            + [pltpu.VMEM((CHUNK,), jnp.int32) for _ in range(2)]
            + [pltpu.SemaphoreType.DMA for _ in range(4)]
        ),
        compiler_params=params,
    )
    out = lookup(idx3, tableP)
    return out.reshape(n_x_rows, row_len, D_MODEL)


# speedup vs baseline: 2.4493x; 1.0008x over previous
import jax
import jax.numpy as jnp
from jax import lax
from jax.experimental import pallas as pl
from jax.experimental.pallas import tpu as pltpu
from jax.experimental.pallas import tpu_sc as plsc

D_MODEL = 64
SCALE = 8.0
CHUNK = 128
LANES = 16
NW = 32


def _wid():
    return lax.axis_index("s") * 2 + lax.axis_index("c")


def _lookup_body(idx_hbm, tableP_hbm, out_hbm, idx_v, gbufs, obufs, pairbufs,
                 gsems, osems, half):
    wid = _wid()
    n_chunks = idx_hbm.shape[1]
    base = wid * n_chunks * CHUNK

    pltpu.sync_copy(idx_hbm.at[wid], idx_v)

    def prep_pair(c, b):
        for k in range(CHUNK // LANES):
            v = idx_v[c, pl.ds(LANES * k, LANES)]
            pairbufs[b][pl.ds(LANES * k, LANES)] = jnp.where(
                v >= half, v - half, v)

    def gather(b):
        return pltpu.make_async_copy(tableP_hbm.at[pairbufs[b]], gbufs[b],
                                     gsems[b])

    def copy_out(c, b):
        return pltpu.make_async_copy(
            obufs[b], out_hbm.at[pl.ds(base + c * CHUNK, CHUNK)], osems[b])

    def select(c, b):
        @plsc.parallel_loop(0, CHUNK // LANES, unroll=4)
        def _(k):
            iv = idx_v[c, pl.ds(k * LANES, LANES)]
            offs = jnp.where(iv >= half, D_MODEL, 0).astype(jnp.int32)
            for j in range(LANES):
                r = k * LANES + j
                off = offs[j]
                for l in range(D_MODEL // LANES):
                    obufs[b][r, pl.ds(l * LANES, LANES)] = (
                        gbufs[b][r, pl.ds(off + l * LANES, LANES)])

    for b in range(2):
        prep_pair(b, b)
        gather(b).start()
    for b in range(2):
        gather(b).wait()
        select(b, b)
        copy_out(b, b).start()
        prep_pair(b + 2, b)
        gather(b).start()

    def group(i, carry):
        for b in range(2):
            c = 2 * i + b
            gather(b).wait()
            copy_out(c - 2, b).wait()
            select(c, b)
            copy_out(c, b).start()
            prep_pair(c + 2, b)
            gather(b).start()
        return carry

    lax.fori_loop(1, n_chunks // 2 - 1, group, 0)

    for b in range(2):
        c = n_chunks - 2 + b
        gather(b).wait()
        copy_out(c - 2, b).wait()
        select(c, b)
        copy_out(c, b).start()
    for b in range(2):
        copy_out(n_chunks - 2 + b, b).wait()


def kernel(x, embed_weight):
    n_x_rows, row_len = x.shape
    n = n_x_rows * row_len
    v = embed_weight.shape[0]
    xi = x.astype(jnp.int32)
    idx3 = xi.reshape(NW, n // (NW * CHUNK), CHUNK)
    tableT = embed_weight.T

    mesh = plsc.VectorSubcoreMesh(core_axis_name="c", subcore_axis_name="s")
    params = pltpu.CompilerParams(needs_layout_passes=False)
    blk = 16384
    vp2 = (v // 2 + blk - 1) // blk * blk

    n_blk = vp2 // blk

    def a_tc_body(x1_ref, x2_ref, o_ref):
        eye = jnp.eye(D_MODEL, dtype=jnp.float32) * SCALE
        dims = (((0,), (0,)), ((), ()))
        left = jax.lax.dot_general(x1_ref[...], eye, dims,
                                   preferred_element_type=jnp.float32)
        right = jax.lax.dot_general(x2_ref[...], eye, dims,
                                    preferred_element_type=jnp.float32)
        o_ref[...] = jnp.concatenate([left, right], axis=1)

    last_in_blk = (v - 1) // blk

    repack = pl.pallas_call(
        a_tc_body,
        grid=(n_blk,),
        in_specs=[
            pl.BlockSpec((D_MODEL, blk), lambda i: (0, i)),
            pl.BlockSpec((D_MODEL, blk),
                         lambda i: (0, jnp.minimum(i + n_blk, last_in_blk))),
        ],
        out_specs=pl.BlockSpec((blk, 2 * D_MODEL), lambda i: (i, 0)),
        out_shape=jax.ShapeDtypeStruct((vp2, 2 * D_MODEL), jnp.float32),
    )
    tableP = repack(tableT, tableT)

    def b_body(idx_hbm, tableP_hbm, out_hbm, idx_v, *scratch):
        _lookup_body(idx_hbm, tableP_hbm, out_hbm, idx_v, scratch[0:2],
                     scratch[2:4], scratch[4:6], scratch[6:8], scratch[8:10],
                     vp2)

    lookup = pl.kernel(
        b_body,
        out_type=jax.ShapeDtypeStruct((n, D_MODEL), jnp.float32),
        mesh=mesh,
        scratch_types=(
            [pltpu.VMEM((n // (NW * CHUNK), CHUNK), jnp.int32)]
            + [pltpu.VMEM((CHUNK, 2 * D_MODEL), jnp.float32) for _ in range(2)]
            + [pltpu.VMEM((CHUNK, D_MODEL), jnp.float32) for _ in range(2)]
            + [pltpu.VMEM((CHUNK,), jnp.int32) for _ in range(2)]
            + [pltpu.SemaphoreType.DMA for _ in range(4)]
        ),
        compiler_params=params,
    )
    out = lookup(idx3, tableP)
    return out.reshape(n_x_rows, row_len, D_MODEL)
